# baseline (device time: 13423 ns/iter reference)
import jax
import jax.numpy as jnp
from jax import lax
from jax.experimental import pallas as pl
from jax.experimental.pallas import tpu as pltpu

M = 512
N = 1024
N_HALF = 512
M_HALF = M // 2
CHUNKS = 8
CM = M_HALF // CHUNKS


def kernel(x):

    def body(x_hbm, out_hbm, xv, sbuf1, rbuf1, pbuf,
             sem_l, sem_o, sem_s1, sem_r1, sem_s2, sem_r2):
        my_x = lax.axis_index("x")
        my_y = lax.axis_index("y")
        x_peer = (1 - my_x, my_y)
        y_peer = (my_x, 1 - my_y)

        local = []
        for c in range(CHUNKS):
            cp = pltpu.make_async_copy(
                x_hbm.at[0, pl.ds(my_y * M_HALF + c * CM, CM), :],
                xv.at[pl.ds(c * CM, CM)],
                sem_l.at[c],
            )
            cp.start()
            local.append(cp)

        barrier = pltpu.get_barrier_semaphore()
        for nbr in (x_peer, y_peer):
            pl.semaphore_signal(
                barrier, inc=1, device_id=nbr,
                device_id_type=pl.DeviceIdType.MESH,
            )
        pl.semaphore_wait(barrier, 2)

        def rows_of(c):
            return pl.ds(my_y * M_HALF + c * CM, CM)

        rdma1 = []
        for c in range(CHUNKS):
            local[c].wait()

            def stage(col0, c=c):
                sbuf1[pl.ds(c * CM, CM)] = (
                    xv[pl.ds(c * CM, CM), col0:col0 + N_HALF]
                    .astype(jnp.bfloat16))

            @pl.when(my_x == 0)
            def _(c=c):
                stage(N_HALF, c)

            @pl.when(my_x == 1)
            def _(c=c):
                stage(0, c)

            r = pltpu.make_async_remote_copy(
                src_ref=sbuf1.at[pl.ds(c * CM, CM)],
                dst_ref=rbuf1.at[pl.ds(c * CM, CM)],
                send_sem=sem_s1.at[c], recv_sem=sem_r1.at[c],
                device_id=x_peer, device_id_type=pl.DeviceIdType.MESH,
            )
            r.start()
            rdma1.append(r)

        rdma2 = []
        ocp = []
        for c in range(CHUNKS):
            rdma1[c].wait_recv()

            def reduce_chunk(col0, c=c):
                own = xv[pl.ds(c * CM, CM), col0:col0 + N_HALF]
                pbuf[pl.ds(c * CM, CM)] = (
                    own + rbuf1[pl.ds(c * CM, CM)].astype(jnp.float32)
                ).astype(jnp.bfloat16)

            @pl.when(my_x == 0)
            def _(c=c):
                reduce_chunk(0, c)

            @pl.when(my_x == 1)
            def _(c=c):
                reduce_chunk(N_HALF, c)

            r = pltpu.make_async_remote_copy(
                src_ref=pbuf.at[pl.ds(c * CM, CM)],
                dst_ref=out_hbm.at[rows_of(c)],
                send_sem=sem_s2.at[c], recv_sem=sem_r2.at[c],
                device_id=y_peer, device_id_type=pl.DeviceIdType.MESH,
            )
            r.start()
            rdma2.append(r)

            cp = pltpu.make_async_copy(
                pbuf.at[pl.ds(c * CM, CM)],
                out_hbm.at[rows_of(c)],
                sem_o.at[c],
            )
            cp.start()
            ocp.append(cp)

        for c in range(CHUNKS):
            rdma2[c].wait_recv()
        for c in range(CHUNKS):
            ocp[c].wait()
            rdma1[c].wait_send()
            rdma2[c].wait_send()

    return pl.pallas_call(
        body,
        out_shape=jax.ShapeDtypeStruct((M, N_HALF), jnp.bfloat16),
        in_specs=[pl.BlockSpec(memory_space=pltpu.MemorySpace.HBM)],
        out_specs=pl.BlockSpec(memory_space=pltpu.MemorySpace.HBM),
        scratch_shapes=[
            pltpu.VMEM((M_HALF, N), jnp.float32),
            pltpu.VMEM((M_HALF, N_HALF), jnp.bfloat16),
            pltpu.VMEM((M_HALF, N_HALF), jnp.bfloat16),
            pltpu.VMEM((M_HALF, N_HALF), jnp.bfloat16),
            pltpu.SemaphoreType.DMA((CHUNKS,)),
            pltpu.SemaphoreType.DMA((CHUNKS,)),
            pltpu.SemaphoreType.DMA((CHUNKS,)),
            pltpu.SemaphoreType.DMA((CHUNKS,)),
            pltpu.SemaphoreType.DMA((CHUNKS,)),
            pltpu.SemaphoreType.DMA((CHUNKS,)),
        ],
        compiler_params=pltpu.CompilerParams(collective_id=0),
    )(x)


# device time: 12253 ns/iter; 1.0955x vs baseline; 1.0955x over previous
import jax
import jax.numpy as jnp
from jax import lax
from jax.experimental import pallas as pl
from jax.experimental.pallas import tpu as pltpu

M = 512
N = 1024
N_HALF = 512
M_HALF = M // 2
CHUNKS = 4
CM = M_HALF // CHUNKS


def kernel(x):

    def body(x_hbm, out_hbm, xv, sbuf1, rbuf1, pbuf,
             sem_l, sem_o, sem_s1, sem_r1, sem_s2, sem_r2):
        my_x = lax.axis_index("x")
        my_y = lax.axis_index("y")
        x_peer = (1 - my_x, my_y)
        y_peer = (my_x, 1 - my_y)

        local = []
        for c in range(CHUNKS):
            cp = pltpu.make_async_copy(
                x_hbm.at[0, pl.ds(my_y * M_HALF + c * CM, CM), :],
                xv.at[pl.ds(c * CM, CM)],
                sem_l.at[c],
            )
            cp.start()
            local.append(cp)

        barrier = pltpu.get_barrier_semaphore()
        for nbr in (x_peer, y_peer):
            pl.semaphore_signal(
                barrier, inc=1, device_id=nbr,
                device_id_type=pl.DeviceIdType.MESH,
            )
        pl.semaphore_wait(barrier, 2)

        def rows_of(c):
            return pl.ds(my_y * M_HALF + c * CM, CM)

        rdma1 = []
        for c in range(CHUNKS):
            local[c].wait()

            def stage(col0, c=c):
                sbuf1[pl.ds(c * CM, CM)] = (
                    xv[pl.ds(c * CM, CM), col0:col0 + N_HALF]
                    .astype(jnp.bfloat16))

            @pl.when(my_x == 0)
            def _(c=c):
                stage(N_HALF, c)

            @pl.when(my_x == 1)
            def _(c=c):
                stage(0, c)

            r = pltpu.make_async_remote_copy(
                src_ref=sbuf1.at[pl.ds(c * CM, CM)],
                dst_ref=rbuf1.at[pl.ds(c * CM, CM)],
                send_sem=sem_s1.at[c], recv_sem=sem_r1.at[c],
                device_id=x_peer, device_id_type=pl.DeviceIdType.MESH,
            )
            r.start()
            rdma1.append(r)

        rdma2 = []
        ocp = []
        for c in range(CHUNKS):
            rdma1[c].wait_recv()

            def reduce_chunk(col0, c=c):
                own = xv[pl.ds(c * CM, CM), col0:col0 + N_HALF]
                pbuf[pl.ds(c * CM, CM)] = (
                    own + rbuf1[pl.ds(c * CM, CM)].astype(jnp.float32)
                ).astype(jnp.bfloat16)

            @pl.when(my_x == 0)
            def _(c=c):
                reduce_chunk(0, c)

            @pl.when(my_x == 1)
            def _(c=c):
                reduce_chunk(N_HALF, c)

            r = pltpu.make_async_remote_copy(
                src_ref=pbuf.at[pl.ds(c * CM, CM)],
                dst_ref=out_hbm.at[rows_of(c)],
                send_sem=sem_s2.at[c], recv_sem=sem_r2.at[c],
                device_id=y_peer, device_id_type=pl.DeviceIdType.MESH,
            )
            r.start()
            rdma2.append(r)

            cp = pltpu.make_async_copy(
                pbuf.at[pl.ds(c * CM, CM)],
                out_hbm.at[rows_of(c)],
                sem_o.at[c],
            )
            cp.start()
            ocp.append(cp)

        for c in range(CHUNKS):
            rdma2[c].wait_recv()
        for c in range(CHUNKS):
            ocp[c].wait()
            rdma1[c].wait_send()
            rdma2[c].wait_send()

    return pl.pallas_call(
        body,
        out_shape=jax.ShapeDtypeStruct((M, N_HALF), jnp.bfloat16),
        in_specs=[pl.BlockSpec(memory_space=pltpu.MemorySpace.HBM)],
        out_specs=pl.BlockSpec(memory_space=pltpu.MemorySpace.HBM),
        scratch_shapes=[
            pltpu.VMEM((M_HALF, N), jnp.float32),
            pltpu.VMEM((M_HALF, N_HALF), jnp.bfloat16),
            pltpu.VMEM((M_HALF, N_HALF), jnp.bfloat16),
            pltpu.VMEM((M_HALF, N_HALF), jnp.bfloat16),
            pltpu.SemaphoreType.DMA((CHUNKS,)),
            pltpu.SemaphoreType.DMA((CHUNKS,)),
            pltpu.SemaphoreType.DMA((CHUNKS,)),
            pltpu.SemaphoreType.DMA((CHUNKS,)),
            pltpu.SemaphoreType.DMA((CHUNKS,)),
            pltpu.SemaphoreType.DMA((CHUNKS,)),
        ],
        compiler_params=pltpu.CompilerParams(collective_id=0),
    )(x)
